# baseline (speedup 1.0000x reference)
import jax
import jax.numpy as jnp
from jax.experimental import pallas as pl

G = 64


def _gcn_conv(x, src, dst, W, b, n):
    h = x @ W
    deg = jnp.zeros((n,), dtype=h.dtype).at[dst].add(1.0)
    dinv = jax.lax.rsqrt(jnp.clip(deg, 1.0, None))
    norm = dinv[src] * dinv[dst]
    out = jnp.zeros((n, W.shape[1]), dtype=h.dtype).at[dst].add(h[src] * norm[:, None])
    return out + b


def _encode(x, edge_index, W1, b1, W2, b2, W3, b3):
    n = x.shape[0]
    loop = jnp.arange(n, dtype=edge_index.dtype)
    src = jnp.concatenate([edge_index[0], loop])
    dst = jnp.concatenate([edge_index[1], loop])
    h = jax.nn.relu(_gcn_conv(x, src, dst, W1, b1, n))
    h = jax.nn.relu(_gcn_conv(h, src, dst, W2, b2, n))
    h = jax.nn.relu(_gcn_conv(h, src, dst, W3, b3, n))
    return h


def _global_mean_pool(x, batch, num_graphs):
    sums = jax.ops.segment_sum(x, batch, num_segments=num_graphs)
    cnts = jax.ops.segment_sum(jnp.ones((x.shape[0],), dtype=x.dtype), batch, num_segments=num_graphs)
    return sums / jnp.clip(cnts, 1.0, None)[:, None]


def _head_kernel(v_ref, w_ref, b_ref, o_ref):
    o_ref[...] = v_ref[...] @ w_ref[...] + b_ref[0, 0]


def kernel(x_q, edge_index_q, batch_q, x_r, edge_index_r, batch_r,
           W1, b1, W2, b2, W3, b3, fcW, fcb):
    hq = _encode(x_q, edge_index_q, W1, b1, W2, b2, W3, b3)
    hr = _encode(x_r, edge_index_r, W1, b1, W2, b2, W3, b3)
    pq = _global_mean_pool(hq, batch_q, G)
    pr = _global_mean_pool(hr, batch_r, G)
    v = jnp.concatenate([pq, pr], axis=1)
    out = pl.pallas_call(
        _head_kernel,
        out_shape=jax.ShapeDtypeStruct((G, 1), jnp.float32),
    )(v, fcW, fcb.reshape(1, 1))
    return out[:, 0]


# speedup vs baseline: 9.8750x; 9.8750x over previous
import functools

import jax
import jax.numpy as jnp
from jax import lax
from jax.experimental import pallas as pl
from jax.experimental.pallas import tpu as pltpu
from jax.experimental.pallas import tpu_sc as plsc

N = 50000
E = 800000
G = 64
GP = 80
NP = 50176
TILE_N = NP // 16
EB = 6256
EBLK = 128
EPAD = EB * EBLK
NB_TILE = EB // 16
RB = 1568
F32 = jnp.float32

_MESH = plsc.VectorSubcoreMesh(core_axis_name="c", subcore_axis_name="s")
_SC_PARAMS = pltpu.CompilerParams(needs_layout_passes=False,
                                  use_tc_tiling_on_sc=False)


_NCH = 4
_CH = NP // _NCH
_TR = _CH // 16


@functools.partial(
    pl.kernel,
    out_type=jax.ShapeDtypeStruct((2 * NP,), F32),
    mesh=_MESH,
    compiler_params=_SC_PARAMS,
    scratch_types=[
        pltpu.VMEM((EBLK,), jnp.int32),
        pltpu.VMEM((NP,), F32),
        pltpu.VMEM((16 * _TR,), F32),
        pltpu.VMEM((_TR,), F32),
        pltpu.VMEM_SHARED((16 * _CH,), F32),
    ],
)
def _deg_kernel(dst_hbm, deg_hbm, dstbuf, part, slab, redbuf, stage):
    c = lax.axis_index("c")
    s = lax.axis_index("s")
    zero16 = jnp.zeros((16,), F32)
    ones16 = jnp.ones((16,), F32)

    def zbody(i, carry):
        part[pl.ds(i * 16, 16)] = zero16
        return carry
    lax.fori_loop(0, NP // 16, zbody, 0)

    def blk(j, carry):
        pltpu.sync_copy(
            dst_hbm.at[pl.ds((c * EB + s * NB_TILE + j) * EBLK, EBLK)], dstbuf)
        for k in range(EBLK // 16):
            idx = dstbuf[pl.ds(k * 16, 16)]
            plsc.addupdate_scatter(part, [idx], ones16)
        return carry
    lax.fori_loop(0, NB_TILE, blk, 0)

    for ch in range(_NCH):
        pltpu.sync_copy(part.at[pl.ds(ch * _CH, _CH)],
                        stage.at[pl.ds(s * _CH, _CH)])
        plsc.subcore_barrier()
        for r in range(16):
            pltpu.sync_copy(stage.at[pl.ds(r * _CH + s * _TR, _TR)],
                            slab.at[pl.ds(r * _TR, _TR)])

        def red(t, carry):
            a = slab[pl.ds(t * 16, 16)]
            for r in range(1, 16):
                a = a + slab[pl.ds(r * _TR + t * 16, 16)]
            redbuf[pl.ds(t * 16, 16)] = a
            return carry
        lax.fori_loop(0, _TR // 16, red, 0)
        pltpu.sync_copy(
            redbuf, deg_hbm.at[pl.ds(c * NP + ch * _CH + s * _TR, _TR)])
        plsc.subcore_barrier()


def _make_agg(nh):
    out_t = [jax.ShapeDtypeStruct((2, NP, 32), F32)] * nh
    scratch = [
        pltpu.VMEM((EBLK,), jnp.int32),
        pltpu.VMEM((EBLK,), jnp.int32),
        pltpu.VMEM((EBLK, 32), F32),
        pltpu.VMEM_SHARED((NP, 32), F32),
        pltpu.SemaphoreType.DMA,
    ]

    def body(*refs):
        y_refs = refs[:nh]
        src_hbm, dst_hbm, zblk_hbm = refs[nh:nh + 3]
        out_refs = refs[nh + 3:2 * nh + 3]
        srcbuf, dstbuf, rows, acc, sem = refs[2 * nh + 3:]
        c = lax.axis_index("c")
        s = lax.axis_index("s")
        off16 = jnp.full((16,), c * NP, jnp.int32)
        for h in range(nh):
            pltpu.sync_copy(zblk_hbm, acc.at[pl.ds(s * TILE_N, TILE_N)])
            plsc.subcore_barrier()

            def blk(j, carry):
                base = (c * EB + s * NB_TILE + j) * EBLK
                pltpu.sync_copy(src_hbm.at[pl.ds(base, EBLK)], srcbuf)
                pltpu.sync_copy(dst_hbm.at[pl.ds(base, EBLK)], dstbuf)
                for k in range(EBLK // 16):
                    srcbuf[pl.ds(k * 16, 16)] = srcbuf[pl.ds(k * 16, 16)] + off16
                pltpu.async_copy(y_refs[h].at[srcbuf], rows, sem).wait()
                pltpu.sync_copy(rows, acc.at[dstbuf], add=True)
                return carry
            lax.fori_loop(0, NB_TILE, blk, 0)
            plsc.subcore_barrier()
            pltpu.sync_copy(acc.at[pl.ds(s * TILE_N, TILE_N)],
                            out_refs[h].at[c, pl.ds(s * TILE_N, TILE_N)])
            plsc.subcore_barrier()

    return pl.kernel(body, out_type=out_t, mesh=_MESH, scratch_types=scratch,
                     compiler_params=_SC_PARAMS)


_agg2 = _make_agg(2)
_agg1 = _make_agg(1)


_PR = NP // 8


@functools.partial(
    pl.kernel,
    out_type=jax.ShapeDtypeStruct((G,), F32),
    mesh=_MESH,
    compiler_params=_SC_PARAMS,
    scratch_types=[
        pltpu.VMEM((_PR,), F32),
        pltpu.VMEM((_PR,), jnp.int32),
        pltpu.VMEM((GP,), F32),
        pltpu.VMEM((GP,), F32),
        pltpu.VMEM((16 * 2 * GP,), F32),
        pltpu.VMEM((G,), F32),
        pltpu.VMEM((16,), F32),
        pltpu.VMEM_SHARED((16 * 2 * GP,), F32),
    ],
)
def _pool_kernel(z_hbm, batch_hbm, fcb_hbm, out_hbm,
                 zbuf, bbuf, accl, cntl, slab, obuf, fbuf, stage):
    c = lax.axis_index("c")
    s = lax.axis_index("s")
    gg = lax.rem(s, 2)
    ci = lax.div(s, 2)
    zero16 = jnp.zeros((16,), F32)
    ones16 = jnp.ones((16,), F32)
    pltpu.sync_copy(z_hbm.at[pl.ds(gg * NP + ci * _PR, _PR)], zbuf)
    pltpu.sync_copy(batch_hbm.at[pl.ds(gg * NP + ci * _PR, _PR)], bbuf)
    pltpu.sync_copy(fcb_hbm, fbuf)
    for t in range(GP // 16):
        accl[pl.ds(t * 16, 16)] = zero16
        cntl[pl.ds(t * 16, 16)] = zero16

    def it(k, carry):
        b16 = bbuf[pl.ds(k * 16, 16)]
        z16 = zbuf[pl.ds(k * 16, 16)]
        plsc.addupdate_scatter(accl, [b16], z16)
        plsc.addupdate_scatter(cntl, [b16], ones16)
        return carry
    lax.fori_loop(0, _PR // 16, it, 0)

    pltpu.sync_copy(accl, stage.at[pl.ds(s * 2 * GP, GP)])
    pltpu.sync_copy(cntl, stage.at[pl.ds(s * 2 * GP + GP, GP)])
    plsc.subcore_barrier()

    @pl.when(jnp.logical_and(c == 0, s == 0))
    def _():
        pltpu.sync_copy(stage, slab)
        fcb16 = fbuf[...]
        for j in range(G // 16):
            def sl(r):
                return pl.ds(r * 2 * GP + j * 16, 16)

            def slc(r):
                return pl.ds(r * 2 * GP + GP + j * 16, 16)
            aq = slab[sl(0)]
            cq = slab[slc(0)]
            ar = slab[sl(1)]
            cr = slab[slc(1)]
            for r in range(2, 16, 2):
                aq = aq + slab[sl(r)]
                cq = cq + slab[slc(r)]
                ar = ar + slab[sl(r + 1)]
                cr = cr + slab[slc(r + 1)]
            o = aq / jnp.maximum(cq, 1.0) + ar / jnp.maximum(cr, 1.0) + fcb16
            obuf[pl.ds(j * 16, 16)] = o
        pltpu.sync_copy(obuf, out_hbm)


def _prep_body(x_ref, deg_ref, dinv_ref, ylo_ref, yhi_ref):
    dinv = lax.rsqrt(deg_ref[0] + 1.0)
    y = x_ref[0] * dinv
    dinv_ref[0] = dinv
    ylo_ref[0] = y[:, :32]
    yhi_ref[0] = y[:, 32:]


def _stage1_body(slo_ref, shi_ref, ylo_ref, yhi_ref, dinv_ref, w1_ref, b1_ref,
                 w2_ref, olo_ref, ohi_ref):
    dinv = dinv_ref[0]
    zlo = dinv * (slo_ref[0] + ylo_ref[0])
    zhi = dinv * (shi_ref[0] + yhi_ref[0])
    z = jnp.concatenate([zlo, zhi], axis=1)
    h1 = jnp.maximum(jnp.dot(z, w1_ref[...], preferred_element_type=F32)
                     + b1_ref[0], 0.0)
    t2 = jnp.dot(h1, w2_ref[...], preferred_element_type=F32) * dinv
    olo_ref[0] = t2[:, :32]
    ohi_ref[0] = t2[:, 32:]


def _stage2_body(slo_ref, shi_ref, ylo_ref, yhi_ref, dinv_ref, b2_ref, w3_ref,
                 o_ref):
    dinv = dinv_ref[0]
    zlo = dinv * (slo_ref[0] + ylo_ref[0])
    zhi = dinv * (shi_ref[0] + yhi_ref[0])
    h2 = jnp.maximum(jnp.concatenate([zlo, zhi], axis=1) + b2_ref[0], 0.0)
    o_ref[0] = jnp.dot(h2, w3_ref[...], preferred_element_type=F32) * dinv


def _stage3_body(s3_ref, y3_ref, dinv_ref, b3_ref, fcw_ref, z_ref):
    h3 = jnp.maximum(dinv_ref[0] * (s3_ref[0] + y3_ref[0]) + b3_ref[0], 0.0)
    z_ref[0] = jnp.dot(h3, fcw_ref[0], preferred_element_type=F32)


def _node_spec(w):
    return pl.BlockSpec((1, RB, w), lambda g, i: (g, i, 0))


def _full_spec(shape):
    return pl.BlockSpec(shape, lambda g, i: tuple(0 for _ in shape))


_GRID = (2, NP // RB)

_prep = pl.pallas_call(
    _prep_body,
    grid=_GRID,
    in_specs=[_node_spec(64), _node_spec(1)],
    out_specs=[_node_spec(1), _node_spec(32), _node_spec(32)],
    out_shape=[jax.ShapeDtypeStruct((2, NP, 1), F32),
               jax.ShapeDtypeStruct((2, NP, 32), F32),
               jax.ShapeDtypeStruct((2, NP, 32), F32)],
)

_stage1 = pl.pallas_call(
    _stage1_body,
    grid=_GRID,
    in_specs=[_node_spec(32), _node_spec(32), _node_spec(32), _node_spec(32),
              _node_spec(1), _full_spec((64, 128)), _full_spec((1, 128)),
              _full_spec((128, 64))],
    out_specs=[_node_spec(32), _node_spec(32)],
    out_shape=[jax.ShapeDtypeStruct((2, NP, 32), F32),
               jax.ShapeDtypeStruct((2, NP, 32), F32)],
)

_stage2 = pl.pallas_call(
    _stage2_body,
    grid=_GRID,
    in_specs=[_node_spec(32), _node_spec(32), _node_spec(32), _node_spec(32),
              _node_spec(1), _full_spec((1, 64)), _full_spec((64, 32))],
    out_specs=_node_spec(32),
    out_shape=jax.ShapeDtypeStruct((2, NP, 32), F32),
)

_stage3 = pl.pallas_call(
    _stage3_body,
    grid=_GRID,
    in_specs=[_node_spec(32), _node_spec(32), _node_spec(1),
              _full_spec((1, 32)),
              pl.BlockSpec((1, 32, 1), lambda g, i: (g, 0, 0))],
    out_specs=_node_spec(1),
    out_shape=jax.ShapeDtypeStruct((2, NP, 1), F32),
)


def kernel(x_q, edge_index_q, batch_q, x_r, edge_index_r, batch_r,
           W1, b1, W2, b2, W3, b3, fcW, fcb):
    X = jnp.pad(jnp.stack([x_q, x_r]), ((0, 0), (0, NP - N), (0, 0)))
    SRC = jnp.pad(jnp.stack([edge_index_q[0], edge_index_r[0]]),
                  ((0, 0), (0, EPAD - E)), constant_values=NP - 1)
    DST = jnp.pad(jnp.stack([edge_index_q[1], edge_index_r[1]]),
                  ((0, 0), (0, EPAD - E)), constant_values=NP - 1)
    SRC = SRC.reshape(-1)
    DST = DST.reshape(-1)
    BATCH = jnp.pad(jnp.stack([batch_q, batch_r]), ((0, 0), (0, NP - N)),
                    constant_values=G).reshape(-1)
    ZBLK = jnp.zeros((TILE_N, 32), F32)
    FCB = jnp.broadcast_to(fcb, (16,))

    DEG = _deg_kernel(DST)
    DINV, Y1LO, Y1HI = _prep(X, DEG.reshape(2, NP, 1))
    S1LO, S1HI = _agg2(Y1LO.reshape(-1, 32), Y1HI.reshape(-1, 32),
                       SRC, DST, ZBLK)
    Y2LO, Y2HI = _stage1(S1LO, S1HI, Y1LO, Y1HI, DINV, W1,
                         b1.reshape(1, 128), W2)
    S2LO, S2HI = _agg2(Y2LO.reshape(-1, 32), Y2HI.reshape(-1, 32),
                       SRC, DST, ZBLK)
    Y3 = _stage2(S2LO, S2HI, Y2LO, Y2HI, DINV, b2.reshape(1, 64), W3)
    (S3,) = _agg1(Y3.reshape(-1, 32), SRC, DST, ZBLK)
    Z = _stage3(S3, Y3, DINV, b3.reshape(1, 32), fcW.reshape(2, 32, 1))
    return _pool_kernel(Z.reshape(-1), BATCH, FCB)


# speedup vs baseline: 21.4982x; 2.1770x over previous
import functools

import jax
import jax.numpy as jnp
from jax import lax
from jax.experimental import pallas as pl
from jax.experimental.pallas import tpu as pltpu
from jax.experimental.pallas import tpu_sc as plsc

N = 50000
E = 800000
G = 64
GP = 80
NP = 50176
TILE_N = NP // 16
EB = 6272
EBLK = 128
EPAD = EB * EBLK
NB_TILE = EB // 16
DPIPE = 4
NGRP = NB_TILE // DPIPE
RB = 1568
F32 = jnp.float32

_MESH = plsc.VectorSubcoreMesh(core_axis_name="c", subcore_axis_name="s")
_SC_PARAMS = pltpu.CompilerParams(needs_layout_passes=False,
                                  use_tc_tiling_on_sc=False)


@functools.partial(
    pl.kernel,
    out_type=[jax.ShapeDtypeStruct((2 * NP,), F32),
              jax.ShapeDtypeStruct((32 * NP,), F32)],
    mesh=_MESH,
    compiler_params=_SC_PARAMS,
    scratch_types=[
        pltpu.VMEM((EBLK,), jnp.int32),
        pltpu.VMEM((NP,), F32),
        pltpu.VMEM((16 * TILE_N,), F32),
        pltpu.VMEM((TILE_N,), F32),
    ],
)
def _deg_kernel(dst_hbm, deg_hbm, scr_hbm, dstbuf, part, slab, redbuf):
    c = lax.axis_index("c")
    s = lax.axis_index("s")
    zero16 = jnp.zeros((16,), F32)
    ones16 = jnp.ones((16,), F32)

    def zbody(i, carry):
        part[pl.ds(i * 16, 16)] = zero16
        return carry
    lax.fori_loop(0, NP // 16, zbody, 0)

    def blk(j, carry):
        pltpu.sync_copy(
            dst_hbm.at[pl.ds((c * EB + s * NB_TILE + j) * EBLK, EBLK)], dstbuf)
        for k in range(EBLK // 16):
            idx = dstbuf[pl.ds(k * 16, 16)]
            plsc.addupdate_scatter(part, [idx], ones16)
        return carry
    lax.fori_loop(0, NB_TILE, blk, 0)

    pltpu.sync_copy(part, scr_hbm.at[pl.ds((c * 16 + s) * NP, NP)])
    plsc.subcore_barrier()
    for r in range(16):
        pltpu.sync_copy(scr_hbm.at[pl.ds((c * 16 + r) * NP + s * TILE_N,
                                         TILE_N)],
                        slab.at[pl.ds(r * TILE_N, TILE_N)])

    def red(t, carry):
        a = slab[pl.ds(t * 16, 16)]
        for r in range(1, 16):
            a = a + slab[pl.ds(r * TILE_N + t * 16, 16)]
        redbuf[pl.ds(t * 16, 16)] = a
        return carry
    lax.fori_loop(0, TILE_N // 16, red, 0)
    pltpu.sync_copy(redbuf, deg_hbm.at[pl.ds(c * NP + s * TILE_N, TILE_N)])


def _make_agg(nh):
    out_t = [jax.ShapeDtypeStruct((2, NP, 32), F32)] * nh
    scratch = [
        pltpu.VMEM((DPIPE, EBLK), jnp.int32),
        pltpu.VMEM((DPIPE, EBLK), jnp.int32),
        pltpu.VMEM((DPIPE, EBLK, 32), F32),
        pltpu.VMEM_SHARED((NP, 32), F32),
        pltpu.SemaphoreType.DMA,
        pltpu.SemaphoreType.DMA,
        pltpu.SemaphoreType.DMA,
    ]

    def body(*refs):
        y_refs = refs[:nh]
        src_hbm, dst_hbm, zblk_hbm = refs[nh:nh + 3]
        out_refs = refs[nh + 3:2 * nh + 3]
        srcb, dstb, rows, acc, isem, gsem, ssem = refs[2 * nh + 3:]
        c = lax.axis_index("c")
        s = lax.axis_index("s")
        off16 = jnp.full((16,), c * NP, jnp.int32)

        def ebase(j):
            return (c * EB + s * NB_TILE + j) * EBLK

        def start_idx(j, b):
            pltpu.async_copy(src_hbm.at[pl.ds(ebase(j), EBLK)],
                             srcb.at[b], isem)
            pltpu.async_copy(dst_hbm.at[pl.ds(ebase(j), EBLK)],
                             dstb.at[b], isem)

        for h in range(nh):
            y = y_refs[h]
            pltpu.sync_copy(zblk_hbm, acc.at[pl.ds(s * TILE_N, TILE_N)])
            plsc.subcore_barrier()
            for b in range(DPIPE):
                start_idx(b, b)

            def grp(g, carry):
                jb = g * DPIPE
                for b in range(DPIPE):
                    pltpu.make_async_copy(src_hbm.at[pl.ds(0, EBLK)],
                                          srcb.at[b], isem).wait()
                    pltpu.make_async_copy(dst_hbm.at[pl.ds(0, EBLK)],
                                          dstb.at[b], isem).wait()
                    for k in range(EBLK // 16):
                        srcb[b, pl.ds(k * 16, 16)] = (
                            srcb[b, pl.ds(k * 16, 16)] + off16)
                    pltpu.async_copy(y.at[srcb.at[b]], rows.at[b], gsem)
                def scat(b, carry2):
                    pltpu.make_async_copy(y.at[pl.ds(0, EBLK)],
                                          rows.at[b], gsem).wait()
                    pltpu.async_copy(rows.at[b], acc.at[dstb.at[b]], ssem,
                                     add=True)
                    return carry2
                lax.fori_loop(0, DPIPE, scat, 0)

                def drain(b, carry2):
                    pltpu.make_async_copy(y.at[pl.ds(0, EBLK)],
                                          rows.at[b], ssem).wait()

                    @pl.when(g < NGRP - 1)
                    def _():
                        start_idx(jb + DPIPE + b, b)
                    return carry2
                lax.fori_loop(0, DPIPE, drain, 0)
                return carry
            lax.fori_loop(0, NGRP, grp, 0)
            plsc.subcore_barrier()
            pltpu.sync_copy(acc.at[pl.ds(s * TILE_N, TILE_N)],
                            out_refs[h].at[c, pl.ds(s * TILE_N, TILE_N)])
            plsc.subcore_barrier()

    return pl.kernel(body, out_type=out_t, mesh=_MESH, scratch_types=scratch,
                     compiler_params=_SC_PARAMS)


_agg2 = _make_agg(2)
_agg1 = _make_agg(1)


_PR = NP // 8


@functools.partial(
    pl.kernel,
    out_type=jax.ShapeDtypeStruct((G,), F32),
    mesh=_MESH,
    compiler_params=_SC_PARAMS,
    scratch_types=[
        pltpu.VMEM((_PR,), F32),
        pltpu.VMEM((_PR,), jnp.int32),
        pltpu.VMEM((GP,), F32),
        pltpu.VMEM((GP,), F32),
        pltpu.VMEM((16 * 2 * GP,), F32),
        pltpu.VMEM((G,), F32),
        pltpu.VMEM((16,), F32),
        pltpu.VMEM_SHARED((16 * 2 * GP,), F32),
    ],
)
def _pool_kernel(z_hbm, batch_hbm, fcb_hbm, out_hbm,
                 zbuf, bbuf, accl, cntl, slab, obuf, fbuf, stage):
    c = lax.axis_index("c")
    s = lax.axis_index("s")
    gg = lax.rem(s, 2)
    ci = lax.div(s, 2)
    zero16 = jnp.zeros((16,), F32)
    ones16 = jnp.ones((16,), F32)
    pltpu.sync_copy(z_hbm.at[pl.ds(gg * NP + ci * _PR, _PR)], zbuf)
    pltpu.sync_copy(batch_hbm.at[pl.ds(gg * NP + ci * _PR, _PR)], bbuf)
    pltpu.sync_copy(fcb_hbm, fbuf)
    for t in range(GP // 16):
        accl[pl.ds(t * 16, 16)] = zero16
        cntl[pl.ds(t * 16, 16)] = zero16

    def it(k, carry):
        b16 = bbuf[pl.ds(k * 16, 16)]
        z16 = zbuf[pl.ds(k * 16, 16)]
        plsc.addupdate_scatter(accl, [b16], z16)
        plsc.addupdate_scatter(cntl, [b16], ones16)
        return carry
    lax.fori_loop(0, _PR // 16, it, 0)

    pltpu.sync_copy(accl, stage.at[pl.ds(s * 2 * GP, GP)])
    pltpu.sync_copy(cntl, stage.at[pl.ds(s * 2 * GP + GP, GP)])
    plsc.subcore_barrier()

    @pl.when(jnp.logical_and(c == 0, s == 0))
    def _():
        pltpu.sync_copy(stage, slab)
        fcb16 = fbuf[...]
        for j in range(G // 16):
            def sl(r):
                return pl.ds(r * 2 * GP + j * 16, 16)

            def slc(r):
                return pl.ds(r * 2 * GP + GP + j * 16, 16)
            aq = slab[sl(0)]
            cq = slab[slc(0)]
            ar = slab[sl(1)]
            cr = slab[slc(1)]
            for r in range(2, 16, 2):
                aq = aq + slab[sl(r)]
                cq = cq + slab[slc(r)]
                ar = ar + slab[sl(r + 1)]
                cr = cr + slab[slc(r + 1)]
            o = aq / jnp.maximum(cq, 1.0) + ar / jnp.maximum(cr, 1.0) + fcb16
            obuf[pl.ds(j * 16, 16)] = o
        pltpu.sync_copy(obuf, out_hbm)


def _prep_body(x_ref, deg_ref, dinv_ref, ylo_ref, yhi_ref):
    dinv = lax.rsqrt(deg_ref[0] + 1.0)
    y = x_ref[0] * dinv
    dinv_ref[0] = dinv
    ylo_ref[0] = y[:, :32]
    yhi_ref[0] = y[:, 32:]


def _stage1_body(slo_ref, shi_ref, ylo_ref, yhi_ref, dinv_ref, w1_ref, b1_ref,
                 w2_ref, olo_ref, ohi_ref):
    dinv = dinv_ref[0]
    zlo = dinv * (slo_ref[0] + ylo_ref[0])
    zhi = dinv * (shi_ref[0] + yhi_ref[0])
    z = jnp.concatenate([zlo, zhi], axis=1)
    h1 = jnp.maximum(jnp.dot(z, w1_ref[...], preferred_element_type=F32)
                     + b1_ref[0], 0.0)
    t2 = jnp.dot(h1, w2_ref[...], preferred_element_type=F32) * dinv
    olo_ref[0] = t2[:, :32]
    ohi_ref[0] = t2[:, 32:]


def _stage2_body(slo_ref, shi_ref, ylo_ref, yhi_ref, dinv_ref, b2_ref, w3_ref,
                 o_ref):
    dinv = dinv_ref[0]
    zlo = dinv * (slo_ref[0] + ylo_ref[0])
    zhi = dinv * (shi_ref[0] + yhi_ref[0])
    h2 = jnp.maximum(jnp.concatenate([zlo, zhi], axis=1) + b2_ref[0], 0.0)
    o_ref[0] = jnp.dot(h2, w3_ref[...], preferred_element_type=F32) * dinv


def _stage3_body(s3_ref, y3_ref, dinv_ref, b3_ref, fcw_ref, z_ref):
    h3 = jnp.maximum(dinv_ref[0] * (s3_ref[0] + y3_ref[0]) + b3_ref[0], 0.0)
    z_ref[0] = jnp.dot(h3, fcw_ref[0], preferred_element_type=F32)


def _node_spec(w):
    return pl.BlockSpec((1, RB, w), lambda g, i: (g, i, 0))


def _full_spec(shape):
    return pl.BlockSpec(shape, lambda g, i: tuple(0 for _ in shape))


_GRID = (2, NP // RB)

_prep = pl.pallas_call(
    _prep_body,
    grid=_GRID,
    in_specs=[_node_spec(64), _node_spec(1)],
    out_specs=[_node_spec(1), _node_spec(32), _node_spec(32)],
    out_shape=[jax.ShapeDtypeStruct((2, NP, 1), F32),
               jax.ShapeDtypeStruct((2, NP, 32), F32),
               jax.ShapeDtypeStruct((2, NP, 32), F32)],
)

_stage1 = pl.pallas_call(
    _stage1_body,
    grid=_GRID,
    in_specs=[_node_spec(32), _node_spec(32), _node_spec(32), _node_spec(32),
              _node_spec(1), _full_spec((64, 128)), _full_spec((1, 128)),
              _full_spec((128, 64))],
    out_specs=[_node_spec(32), _node_spec(32)],
    out_shape=[jax.ShapeDtypeStruct((2, NP, 32), F32),
               jax.ShapeDtypeStruct((2, NP, 32), F32)],
)

_stage2 = pl.pallas_call(
    _stage2_body,
    grid=_GRID,
    in_specs=[_node_spec(32), _node_spec(32), _node_spec(32), _node_spec(32),
              _node_spec(1), _full_spec((1, 64)), _full_spec((64, 32))],
    out_specs=_node_spec(32),
    out_shape=jax.ShapeDtypeStruct((2, NP, 32), F32),
)

_stage3 = pl.pallas_call(
    _stage3_body,
    grid=_GRID,
    in_specs=[_node_spec(32), _node_spec(32), _node_spec(1),
              _full_spec((1, 32)),
              pl.BlockSpec((1, 32, 1), lambda g, i: (g, 0, 0))],
    out_specs=_node_spec(1),
    out_shape=jax.ShapeDtypeStruct((2, NP, 1), F32),
)


def kernel(x_q, edge_index_q, batch_q, x_r, edge_index_r, batch_r,
           W1, b1, W2, b2, W3, b3, fcW, fcb):
    X = jnp.pad(jnp.stack([x_q, x_r]), ((0, 0), (0, NP - N), (0, 0)))
    SRC = jnp.pad(jnp.stack([edge_index_q[0], edge_index_r[0]]),
                  ((0, 0), (0, EPAD - E)), constant_values=NP - 1)
    DST = jnp.pad(jnp.stack([edge_index_q[1], edge_index_r[1]]),
                  ((0, 0), (0, EPAD - E)), constant_values=NP - 1)
    SRC = SRC.reshape(-1)
    DST = DST.reshape(-1)
    BATCH = jnp.pad(jnp.stack([batch_q, batch_r]), ((0, 0), (0, NP - N)),
                    constant_values=G).reshape(-1)
    ZBLK = jnp.zeros((TILE_N, 32), F32)
    FCB = jnp.broadcast_to(fcb, (16,))

    DEG, _ = _deg_kernel(DST)
    DINV, Y1LO, Y1HI = _prep(X, DEG.reshape(2, NP, 1))
    S1LO, S1HI = _agg2(Y1LO.reshape(-1, 32), Y1HI.reshape(-1, 32),
                       SRC, DST, ZBLK)
    Y2LO, Y2HI = _stage1(S1LO, S1HI, Y1LO, Y1HI, DINV, W1,
                         b1.reshape(1, 128), W2)
    S2LO, S2HI = _agg2(Y2LO.reshape(-1, 32), Y2HI.reshape(-1, 32),
                       SRC, DST, ZBLK)
    Y3 = _stage2(S2LO, S2HI, Y2LO, Y2HI, DINV, b2.reshape(1, 64), W3)
    (S3,) = _agg1(Y3.reshape(-1, 32), SRC, DST, ZBLK)
    Z = _stage3(S3, Y3, DINV, b3.reshape(1, 32), fcW.reshape(2, 32, 1))
    return _pool_kernel(Z.reshape(-1), BATCH, FCB)


# speedup vs baseline: 22.5757x; 1.0501x over previous
import functools

import jax
import jax.numpy as jnp
from jax import lax
from jax.experimental import pallas as pl
from jax.experimental.pallas import tpu as pltpu
from jax.experimental.pallas import tpu_sc as plsc

N = 50000
E = 800000
G = 64
GP = 80
NP = 50176
NP2 = 2 * NP
TILE_N = NP // 16
EB = 6272
EBLK = 128
EPAD = EB * EBLK
NB_TILE = EB // 16
DPIPE = 4
NGRP = NB_TILE // DPIPE
RB = 1568
NBLK = NP // RB
F32 = jnp.float32

_MESH = plsc.VectorSubcoreMesh(core_axis_name="c", subcore_axis_name="s")
_SC_PARAMS = pltpu.CompilerParams(needs_layout_passes=False,
                                  use_tc_tiling_on_sc=False)


@functools.partial(
    pl.kernel,
    out_type=jax.ShapeDtypeStruct((NP2, 32), F32),
    mesh=_MESH,
    compiler_params=_SC_PARAMS,
    scratch_types=[
        pltpu.VMEM((DPIPE, EBLK), jnp.int32),
        pltpu.VMEM((EBLK, 32), F32),
        pltpu.VMEM_SHARED((NP, 32), F32),
        pltpu.SemaphoreType.DMA,
        pltpu.SemaphoreType.DMA,
    ],
)
def _degb_kernel(dst_hbm, onesb_hbm, zblk_hbm, degb_hbm,
                 dstb, ones, acc, isem, ssem):
    c = lax.axis_index("c")
    s = lax.axis_index("s")
    pltpu.sync_copy(onesb_hbm, ones)
    pltpu.sync_copy(zblk_hbm, acc.at[pl.ds(s * TILE_N, TILE_N)])
    plsc.subcore_barrier()

    def start_idx(j, b):
        pltpu.async_copy(
            dst_hbm.at[pl.ds((c * EB + s * NB_TILE + j) * EBLK, EBLK)],
            dstb.at[b], isem)

    for b in range(DPIPE):
        start_idx(b, b)

    def grp(g, carry):
        def seg(b, carry2):
            pltpu.make_async_copy(dst_hbm.at[pl.ds(0, EBLK)],
                                  dstb.at[b], isem).wait()
            pltpu.async_copy(ones, acc.at[dstb.at[b]], ssem, add=True)
            return carry2
        lax.fori_loop(0, DPIPE, seg, 0)

        def drain(b, carry2):
            pltpu.make_async_copy(onesb_hbm, ones, ssem).wait()

            @pl.when(g < NGRP - 1)
            def _():
                start_idx(g * DPIPE + DPIPE + b, b)
            return carry2
        lax.fori_loop(0, DPIPE, drain, 0)
        return carry
    lax.fori_loop(0, NGRP, grp, 0)
    plsc.subcore_barrier()
    pltpu.sync_copy(acc.at[pl.ds(s * TILE_N, TILE_N)],
                    degb_hbm.at[pl.ds(c * NP + s * TILE_N, TILE_N)])


def _make_agg(nh):
    out_t = [jax.ShapeDtypeStruct((NP2, 32), F32)] * nh
    scratch = [
        pltpu.VMEM((DPIPE, EBLK), jnp.int32),
        pltpu.VMEM((DPIPE, EBLK), jnp.int32),
        pltpu.VMEM((DPIPE, EBLK, 32), F32),
        pltpu.VMEM_SHARED((NP, 32), F32),
        pltpu.SemaphoreType.DMA,
        pltpu.SemaphoreType.DMA,
        pltpu.SemaphoreType.DMA,
    ]

    def body(*refs):
        y_refs = refs[:nh]
        src_hbm, dst_hbm, zblk_hbm = refs[nh:nh + 3]
        out_refs = refs[nh + 3:2 * nh + 3]
        srcb, dstb, rows, acc, isem, gsem, ssem = refs[2 * nh + 3:]
        c = lax.axis_index("c")
        s = lax.axis_index("s")
        off16 = jnp.full((16,), c * NP, jnp.int32)

        def ebase(j):
            return (c * EB + s * NB_TILE + j) * EBLK

        def start_idx(j, b):
            pltpu.async_copy(src_hbm.at[pl.ds(ebase(j), EBLK)],
                             srcb.at[b], isem)
            pltpu.async_copy(dst_hbm.at[pl.ds(ebase(j), EBLK)],
                             dstb.at[b], isem)

        for h in range(nh):
            y = y_refs[h]
            pltpu.sync_copy(zblk_hbm, acc.at[pl.ds(s * TILE_N, TILE_N)])
            plsc.subcore_barrier()
            for b in range(DPIPE):
                start_idx(b, b)

            def grp(g, carry):
                jb = g * DPIPE
                for b in range(DPIPE):
                    pltpu.make_async_copy(src_hbm.at[pl.ds(0, EBLK)],
                                          srcb.at[b], isem).wait()
                    pltpu.make_async_copy(dst_hbm.at[pl.ds(0, EBLK)],
                                          dstb.at[b], isem).wait()
                    for k in range(EBLK // 16):
                        srcb[b, pl.ds(k * 16, 16)] = (
                            srcb[b, pl.ds(k * 16, 16)] + off16)
                    pltpu.async_copy(y.at[srcb.at[b]], rows.at[b], gsem)

                def scat(b, carry2):
                    pltpu.make_async_copy(y.at[pl.ds(0, EBLK)],
                                          rows.at[b], gsem).wait()
                    pltpu.async_copy(rows.at[b], acc.at[dstb.at[b]], ssem,
                                     add=True)
                    return carry2
                lax.fori_loop(0, DPIPE, scat, 0)

                def drain(b, carry2):
                    pltpu.make_async_copy(y.at[pl.ds(0, EBLK)],
                                          rows.at[b], ssem).wait()

                    @pl.when(g < NGRP - 1)
                    def _():
                        start_idx(jb + DPIPE + b, b)
                    return carry2
                lax.fori_loop(0, DPIPE, drain, 0)
                return carry
            lax.fori_loop(0, NGRP, grp, 0)
            plsc.subcore_barrier()
            pltpu.sync_copy(acc.at[pl.ds(s * TILE_N, TILE_N)],
                            out_refs[h].at[pl.ds(c * NP + s * TILE_N, TILE_N)])
            plsc.subcore_barrier()

    return pl.kernel(body, out_type=out_t, mesh=_MESH, scratch_types=scratch,
                     compiler_params=_SC_PARAMS)


_agg2 = _make_agg(2)
_agg1 = _make_agg(1)


_PR = NP // 8
_CHZ = 448
_NCHZ = _PR // _CHZ


@functools.partial(
    pl.kernel,
    out_type=jax.ShapeDtypeStruct((G,), F32),
    mesh=_MESH,
    compiler_params=_SC_PARAMS,
    scratch_types=[
        pltpu.VMEM((_CHZ, 32), F32),
        pltpu.VMEM((_PR,), jnp.int32),
        pltpu.VMEM((GP,), F32),
        pltpu.VMEM((GP,), F32),
        pltpu.VMEM((16 * 2 * GP,), F32),
        pltpu.VMEM((G,), F32),
        pltpu.VMEM((16,), F32),
        pltpu.VMEM_SHARED((16 * 2 * GP,), F32),
    ],
)
def _pool_kernel(z_hbm, batch_hbm, fcb_hbm, out_hbm,
                 zbuf, bbuf, accl, cntl, slab, obuf, fbuf, stage):
    c = lax.axis_index("c")
    s = lax.axis_index("s")
    gg = lax.rem(s, 2)
    ci = lax.div(s, 2)
    zero16 = jnp.zeros((16,), F32)
    ones16 = jnp.ones((16,), F32)
    iota16 = lax.iota(jnp.int32, 16)
    zero16i = jnp.zeros((16,), jnp.int32)
    pltpu.sync_copy(batch_hbm.at[pl.ds(gg * NP + ci * _PR, _PR)], bbuf)
    pltpu.sync_copy(fcb_hbm, fbuf)
    for t in range(GP // 16):
        accl[pl.ds(t * 16, 16)] = zero16
        cntl[pl.ds(t * 16, 16)] = zero16

    def chunk(ch, carry):
        pltpu.sync_copy(z_hbm.at[pl.ds(gg * NP + ci * _PR + ch * _CHZ, _CHZ)],
                        zbuf)

        def it(k, carry2):
            b16 = bbuf[pl.ds(ch * _CHZ + k * 16, 16)]
            rows16 = iota16 + k * 16
            z16 = plsc.load_gather(zbuf, [rows16, zero16i])
            plsc.addupdate_scatter(accl, [b16], z16)
            plsc.addupdate_scatter(cntl, [b16], ones16)
            return carry2
        lax.fori_loop(0, _CHZ // 16, it, 0)
        return carry
    lax.fori_loop(0, _NCHZ, chunk, 0)

    pltpu.sync_copy(accl, stage.at[pl.ds(s * 2 * GP, GP)])
    pltpu.sync_copy(cntl, stage.at[pl.ds(s * 2 * GP + GP, GP)])
    plsc.subcore_barrier()

    @pl.when(jnp.logical_and(c == 0, s == 0))
    def _():
        pltpu.sync_copy(stage, slab)
        fcb16 = fbuf[...]
        for j in range(G // 16):
            def sl(r):
                return pl.ds(r * 2 * GP + j * 16, 16)

            def slc(r):
                return pl.ds(r * 2 * GP + GP + j * 16, 16)
            aq = slab[sl(0)]
            cq = slab[slc(0)]
            ar = slab[sl(1)]
            cr = slab[slc(1)]
            for r in range(2, 16, 2):
                aq = aq + slab[sl(r)]
                cq = cq + slab[slc(r)]
                ar = ar + slab[sl(r + 1)]
                cr = cr + slab[slc(r + 1)]
            o = aq / jnp.maximum(cq, 1.0) + ar / jnp.maximum(cr, 1.0) + fcb16
            obuf[pl.ds(j * 16, 16)] = o
        pltpu.sync_copy(obuf, out_hbm)


def _dinv(degb):
    return lax.rsqrt(degb + 1.0)


def _prep_body(x_ref, degb_ref, ylo_ref, yhi_ref):
    dinv = _dinv(degb_ref[...])
    ylo_ref[...] = x_ref[:, :32] * dinv
    yhi_ref[...] = x_ref[:, 32:] * dinv


def _stage1_body(slo_ref, shi_ref, ylo_ref, yhi_ref, degb_ref, w1_ref, b1_ref,
                 w2_ref, olo_ref, ohi_ref):
    dinv = _dinv(degb_ref[...])
    zlo = dinv * (slo_ref[...] + ylo_ref[...])
    zhi = dinv * (shi_ref[...] + yhi_ref[...])
    z = jnp.concatenate([zlo, zhi], axis=1)
    h1 = jnp.maximum(jnp.dot(z, w1_ref[...], preferred_element_type=F32)
                     + b1_ref[0], 0.0)
    t2 = jnp.dot(h1, w2_ref[...], preferred_element_type=F32)
    olo_ref[...] = t2[:, :32] * dinv
    ohi_ref[...] = t2[:, 32:] * dinv


def _stage2_body(slo_ref, shi_ref, ylo_ref, yhi_ref, degb_ref, b2_ref, w3_ref,
                 o_ref):
    dinv = _dinv(degb_ref[...])
    zlo = dinv * (slo_ref[...] + ylo_ref[...])
    zhi = dinv * (shi_ref[...] + yhi_ref[...])
    h2 = jnp.maximum(jnp.concatenate([zlo, zhi], axis=1) + b2_ref[0], 0.0)
    o_ref[...] = jnp.dot(h2, w3_ref[...], preferred_element_type=F32) * dinv


def _stage3_body(s3_ref, y3_ref, degb_ref, b3_ref, fcw_ref, z_ref):
    dinv = _dinv(degb_ref[...])
    h3 = jnp.maximum(dinv * (s3_ref[...] + y3_ref[...]) + b3_ref[0], 0.0)
    z = jnp.dot(h3, fcw_ref[0], preferred_element_type=F32)
    z_ref[...] = jnp.broadcast_to(z, (RB, 32))


def _node_spec(w):
    return pl.BlockSpec((RB, w), lambda g, i: (g * NBLK + i, 0))


def _full_spec(shape):
    return pl.BlockSpec(shape, lambda g, i: tuple(0 for _ in shape))


_GRID = (2, NBLK)


def _nsd(w=32):
    return jax.ShapeDtypeStruct((NP2, w), F32)


_prep = pl.pallas_call(
    _prep_body,
    grid=_GRID,
    in_specs=[_node_spec(64), _node_spec(32)],
    out_specs=[_node_spec(32), _node_spec(32)],
    out_shape=[_nsd(), _nsd()],
)

_stage1 = pl.pallas_call(
    _stage1_body,
    grid=_GRID,
    in_specs=[_node_spec(32), _node_spec(32), _node_spec(32), _node_spec(32),
              _node_spec(32), _full_spec((64, 128)), _full_spec((1, 128)),
              _full_spec((128, 64))],
    out_specs=[_node_spec(32), _node_spec(32)],
    out_shape=[_nsd(), _nsd()],
)

_stage2 = pl.pallas_call(
    _stage2_body,
    grid=_GRID,
    in_specs=[_node_spec(32), _node_spec(32), _node_spec(32), _node_spec(32),
              _node_spec(32), _full_spec((1, 64)), _full_spec((64, 32))],
    out_specs=_node_spec(32),
    out_shape=_nsd(),
)

_stage3 = pl.pallas_call(
    _stage3_body,
    grid=_GRID,
    in_specs=[_node_spec(32), _node_spec(32), _node_spec(32),
              _full_spec((1, 32)),
              pl.BlockSpec((1, 32, 1), lambda g, i: (g, 0, 0))],
    out_specs=_node_spec(32),
    out_shape=_nsd(),
)


def kernel(x_q, edge_index_q, batch_q, x_r, edge_index_r, batch_r,
           W1, b1, W2, b2, W3, b3, fcW, fcb):
    X = jnp.pad(jnp.stack([x_q, x_r]),
                ((0, 0), (0, NP - N), (0, 0))).reshape(NP2, 64)
    SRC = jnp.pad(jnp.stack([edge_index_q[0], edge_index_r[0]]),
                  ((0, 0), (0, EPAD - E)), constant_values=NP - 1).reshape(-1)
    DST = jnp.pad(jnp.stack([edge_index_q[1], edge_index_r[1]]),
                  ((0, 0), (0, EPAD - E)), constant_values=NP - 1).reshape(-1)
    BATCH = jnp.pad(jnp.stack([batch_q, batch_r]), ((0, 0), (0, NP - N)),
                    constant_values=G).reshape(-1)
    ZBLK = jnp.zeros((TILE_N, 32), F32)
    ONESB = jnp.ones((EBLK, 32), F32)
    FCB = jnp.broadcast_to(fcb, (16,))

    DEGB = _degb_kernel(DST, ONESB, ZBLK)
    Y1LO, Y1HI = _prep(X, DEGB)
    S1LO, S1HI = _agg2(Y1LO, Y1HI, SRC, DST, ZBLK)
    Y2LO, Y2HI = _stage1(S1LO, S1HI, Y1LO, Y1HI, DEGB, W1,
                         b1.reshape(1, 128), W2)
    S2LO, S2HI = _agg2(Y2LO, Y2HI, SRC, DST, ZBLK)
    Y3 = _stage2(S2LO, S2HI, Y2LO, Y2HI, DEGB, b2.reshape(1, 64), W3)
    (S3,) = _agg1(Y3, SRC, DST, ZBLK)
    ZB = _stage3(S3, Y3, DEGB, b3.reshape(1, 32), fcW.reshape(2, 32, 1))
    return _pool_kernel(ZB, BATCH, FCB)


# speedup vs baseline: 27.0568x; 1.1985x over previous
import functools

import jax
import jax.numpy as jnp
from jax import lax
from jax.experimental import pallas as pl
from jax.experimental.pallas import tpu as pltpu
from jax.experimental.pallas import tpu_sc as plsc

N = 50000
E = 800000
G = 64
GP = 80
NP = 50176
TILE_N = NP // 16
EB = 6272
EBLK = 128
EPAD = EB * EBLK
NBT_ALL = EB // 16
NBT_SPL = EB // 32
DPIPE = 4
RB = 1568
NBLK = NP // RB
F32 = jnp.float32

_MESH = plsc.VectorSubcoreMesh(core_axis_name="c", subcore_axis_name="s")
_SC_PARAMS = pltpu.CompilerParams(needs_layout_passes=False,
                                  use_tc_tiling_on_sc=False)
_NSD = jax.ShapeDtypeStruct((NP, 32), F32)


def _edge_pipeline(idx_start, idx_wait, work, drain_one, nblocks):
    ngrp = nblocks // DPIPE
    for b in range(DPIPE):
        idx_start(b, b)

    def grp(g, carry):
        for b in range(DPIPE):
            idx_wait(b)
            work(b)

        def drain(b, carry2):
            drain_one(b)

            @pl.when(g < ngrp - 1)
            def _():
                idx_start(g * DPIPE + DPIPE + b, b)
            return carry2
        lax.fori_loop(0, DPIPE, drain, 0)
        return carry
    lax.fori_loop(0, ngrp, grp, 0)


@functools.partial(
    pl.kernel,
    out_type=[_NSD, _NSD],
    mesh=_MESH,
    compiler_params=_SC_PARAMS,
    scratch_types=[
        pltpu.VMEM((DPIPE, EBLK), jnp.int32),
        pltpu.VMEM((EBLK, 32), F32),
        pltpu.VMEM_SHARED((NP, 32), F32),
        pltpu.SemaphoreType.DMA,
        pltpu.SemaphoreType.DMA,
    ],
)
def _deg_g(dst_hbm, onesb_hbm, zblk_hbm, oa, ob, dstb, ones, acc, isem, ssem):
    c = lax.axis_index("c")
    s = lax.axis_index("s")
    pltpu.sync_copy(onesb_hbm, ones)
    pltpu.sync_copy(zblk_hbm, acc.at[pl.ds(s * TILE_N, TILE_N)])
    plsc.subcore_barrier()

    def idx_start(j, b):
        pltpu.async_copy(
            dst_hbm.at[pl.ds(((c * 16 + s) * NBT_SPL + j) * EBLK, EBLK)],
            dstb.at[b], isem)

    def idx_wait(b):
        pltpu.make_async_copy(dst_hbm.at[pl.ds(0, EBLK)],
                              dstb.at[b], isem).wait()

    def work(b):
        pltpu.async_copy(ones, acc.at[dstb.at[b]], ssem, add=True)

    def drain_one(b):
        pltpu.make_async_copy(onesb_hbm, ones, ssem).wait()

    _edge_pipeline(idx_start, idx_wait, work, drain_one, NBT_SPL)
    plsc.subcore_barrier()
    for cc, out in ((0, oa), (1, ob)):
        @pl.when(c == cc)
        def _(out=out):
            pltpu.sync_copy(acc.at[pl.ds(s * TILE_N, TILE_N)],
                            out.at[pl.ds(s * TILE_N, TILE_N)])


@functools.partial(
    pl.kernel,
    out_type=[_NSD, _NSD],
    mesh=_MESH,
    compiler_params=_SC_PARAMS,
    scratch_types=[
        pltpu.VMEM((DPIPE, EBLK), jnp.int32),
        pltpu.VMEM((DPIPE, EBLK), jnp.int32),
        pltpu.VMEM((DPIPE, EBLK, 32), F32),
        pltpu.VMEM_SHARED((NP, 32), F32),
        pltpu.SemaphoreType.DMA,
        pltpu.SemaphoreType.DMA,
        pltpu.SemaphoreType.DMA,
    ],
)
def _agg2_g(ylo_hbm, yhi_hbm, src_hbm, dst_hbm, zblk_hbm, olo, ohi,
            srcb, dstb, rows, acc, isem, gsem, ssem):
    c = lax.axis_index("c")
    s = lax.axis_index("s")
    pltpu.sync_copy(zblk_hbm, acc.at[pl.ds(s * TILE_N, TILE_N)])
    plsc.subcore_barrier()

    def idx_start(j, b):
        base = (s * NBT_ALL + j) * EBLK
        pltpu.async_copy(src_hbm.at[pl.ds(base, EBLK)], srcb.at[b], isem)
        pltpu.async_copy(dst_hbm.at[pl.ds(base, EBLK)], dstb.at[b], isem)

    def idx_wait(b):
        pltpu.make_async_copy(src_hbm.at[pl.ds(0, EBLK)],
                              srcb.at[b], isem).wait()
        pltpu.make_async_copy(dst_hbm.at[pl.ds(0, EBLK)],
                              dstb.at[b], isem).wait()

    def drain_one(b):
        pltpu.make_async_copy(ylo_hbm.at[pl.ds(0, EBLK)],
                              rows.at[b], ssem).wait()

    for h, y, out in ((0, ylo_hbm, olo), (1, yhi_hbm, ohi)):
        @pl.when(c == h)
        def _(y=y, out=out):
            def work(b):
                pltpu.async_copy(y.at[srcb.at[b]], rows.at[b], gsem)

                def scat(bb, carry):
                    pltpu.make_async_copy(y.at[pl.ds(0, EBLK)],
                                          rows.at[bb], gsem).wait()
                    pltpu.async_copy(rows.at[bb], acc.at[dstb.at[bb]], ssem,
                                     add=True)
                    return carry
                if b == DPIPE - 1:
                    lax.fori_loop(0, DPIPE, scat, 0)

            _edge_pipeline(idx_start, idx_wait, work, drain_one, NBT_ALL)
            plsc.subcore_barrier()
            pltpu.sync_copy(acc.at[pl.ds(s * TILE_N, TILE_N)],
                            out.at[pl.ds(s * TILE_N, TILE_N)])


@functools.partial(
    pl.kernel,
    out_type=[_NSD, _NSD],
    mesh=_MESH,
    compiler_params=_SC_PARAMS,
    scratch_types=[
        pltpu.VMEM((DPIPE, EBLK), jnp.int32),
        pltpu.VMEM((DPIPE, EBLK), jnp.int32),
        pltpu.VMEM((DPIPE, EBLK, 32), F32),
        pltpu.VMEM_SHARED((NP, 32), F32),
        pltpu.SemaphoreType.DMA,
        pltpu.SemaphoreType.DMA,
        pltpu.SemaphoreType.DMA,
    ],
)
def _agg1_g(y_hbm, src_hbm, dst_hbm, zblk_hbm, oa, ob,
            srcb, dstb, rows, acc, isem, gsem, ssem):
    c = lax.axis_index("c")
    s = lax.axis_index("s")
    pltpu.sync_copy(zblk_hbm, acc.at[pl.ds(s * TILE_N, TILE_N)])
    plsc.subcore_barrier()

    def idx_start(j, b):
        base = ((c * 16 + s) * NBT_SPL + j) * EBLK
        pltpu.async_copy(src_hbm.at[pl.ds(base, EBLK)], srcb.at[b], isem)
        pltpu.async_copy(dst_hbm.at[pl.ds(base, EBLK)], dstb.at[b], isem)

    def idx_wait(b):
        pltpu.make_async_copy(src_hbm.at[pl.ds(0, EBLK)],
                              srcb.at[b], isem).wait()
        pltpu.make_async_copy(dst_hbm.at[pl.ds(0, EBLK)],
                              dstb.at[b], isem).wait()

    def work(b):
        pltpu.async_copy(y_hbm.at[srcb.at[b]], rows.at[b], gsem)

        def scat(bb, carry):
            pltpu.make_async_copy(y_hbm.at[pl.ds(0, EBLK)],
                                  rows.at[bb], gsem).wait()
            pltpu.async_copy(rows.at[bb], acc.at[dstb.at[bb]], ssem, add=True)
            return carry
        if b == DPIPE - 1:
            lax.fori_loop(0, DPIPE, scat, 0)

    def drain_one(b):
        pltpu.make_async_copy(y_hbm.at[pl.ds(0, EBLK)],
                              rows.at[b], ssem).wait()

    _edge_pipeline(idx_start, idx_wait, work, drain_one, NBT_SPL)
    plsc.subcore_barrier()
    for cc, out in ((0, oa), (1, ob)):
        @pl.when(c == cc)
        def _(out=out):
            pltpu.sync_copy(acc.at[pl.ds(s * TILE_N, TILE_N)],
                            out.at[pl.ds(s * TILE_N, TILE_N)])


_PR = NP // 8
_CHZ = 448
_NCHZ = _PR // _CHZ


@functools.partial(
    pl.kernel,
    out_type=jax.ShapeDtypeStruct((G,), F32),
    mesh=_MESH,
    compiler_params=_SC_PARAMS,
    scratch_types=[
        pltpu.VMEM((_CHZ, 32), F32),
        pltpu.VMEM((_PR,), jnp.int32),
        pltpu.VMEM((GP,), F32),
        pltpu.VMEM((GP,), F32),
        pltpu.VMEM((16 * 2 * GP,), F32),
        pltpu.VMEM((G,), F32),
        pltpu.VMEM((16,), F32),
        pltpu.VMEM_SHARED((16 * 2 * GP,), F32),
    ],
)
def _pool_kernel(zq_hbm, zr_hbm, bq_hbm, br_hbm, fcb_hbm, out_hbm,
                 zbuf, bbuf, accl, cntl, slab, obuf, fbuf, stage):
    c = lax.axis_index("c")
    s = lax.axis_index("s")
    gg = lax.rem(s, 2)
    ci = lax.div(s, 2)
    zero16 = jnp.zeros((16,), F32)
    ones16 = jnp.ones((16,), F32)
    iota16 = lax.iota(jnp.int32, 16)
    zero16i = jnp.zeros((16,), jnp.int32)
    for gch, bh in ((0, bq_hbm), (1, br_hbm)):
        @pl.when(gg == gch)
        def _(bh=bh):
            pltpu.sync_copy(bh.at[pl.ds(ci * _PR, _PR)], bbuf)
    pltpu.sync_copy(fcb_hbm, fbuf)
    for t in range(GP // 16):
        accl[pl.ds(t * 16, 16)] = zero16
        cntl[pl.ds(t * 16, 16)] = zero16

    def chunk(ch, carry):
        for gch, zh in ((0, zq_hbm), (1, zr_hbm)):
            @pl.when(gg == gch)
            def _(zh=zh):
                pltpu.sync_copy(zh.at[pl.ds(ci * _PR + ch * _CHZ, _CHZ)], zbuf)

        def it(k, carry2):
            b16 = bbuf[pl.ds(ch * _CHZ + k * 16, 16)]
            rows16 = iota16 + k * 16
            z16 = plsc.load_gather(zbuf, [rows16, zero16i])
            plsc.addupdate_scatter(accl, [b16], z16)
            plsc.addupdate_scatter(cntl, [b16], ones16)
            return carry2
        lax.fori_loop(0, _CHZ // 16, it, 0)
        return carry
    lax.fori_loop(0, _NCHZ, chunk, 0)

    pltpu.sync_copy(accl, stage.at[pl.ds(s * 2 * GP, GP)])
    pltpu.sync_copy(cntl, stage.at[pl.ds(s * 2 * GP + GP, GP)])
    plsc.subcore_barrier()

    @pl.when(jnp.logical_and(c == 0, s == 0))
    def _():
        pltpu.sync_copy(stage, slab)
        fcb16 = fbuf[...]
        for j in range(G // 16):
            def sl(r):
                return pl.ds(r * 2 * GP + j * 16, 16)

            def slc(r):
                return pl.ds(r * 2 * GP + GP + j * 16, 16)
            aq = slab[sl(0)]
            cq = slab[slc(0)]
            ar = slab[sl(1)]
            cr = slab[slc(1)]
            for r in range(2, 16, 2):
                aq = aq + slab[sl(r)]
                cq = cq + slab[slc(r)]
                ar = ar + slab[sl(r + 1)]
                cr = cr + slab[slc(r + 1)]
            o = aq / jnp.maximum(cq, 1.0) + ar / jnp.maximum(cr, 1.0) + fcb16
            obuf[pl.ds(j * 16, 16)] = o
        pltpu.sync_copy(obuf, out_hbm)


def _prep_body(x_ref, da_ref, db_ref, ylo_ref, yhi_ref, dv_ref):
    dinv = lax.rsqrt(da_ref[...] + db_ref[...] + 1.0)
    ylo_ref[...] = x_ref[:, :32] * dinv
    yhi_ref[...] = x_ref[:, 32:] * dinv
    dv_ref[...] = dinv


def _stage1_body(slo_ref, shi_ref, ylo_ref, yhi_ref, dv_ref, w1_ref, b1_ref,
                 w2_ref, olo_ref, ohi_ref):
    dinv = dv_ref[...]
    zlo = dinv * (slo_ref[...] + ylo_ref[...])
    zhi = dinv * (shi_ref[...] + yhi_ref[...])
    z = jnp.concatenate([zlo, zhi], axis=1)
    h1 = jnp.maximum(jnp.dot(z, w1_ref[...], preferred_element_type=F32)
                     + b1_ref[0], 0.0)
    t2 = jnp.dot(h1, w2_ref[...], preferred_element_type=F32)
    olo_ref[...] = t2[:, :32] * dinv
    ohi_ref[...] = t2[:, 32:] * dinv


def _stage2_body(slo_ref, shi_ref, ylo_ref, yhi_ref, dv_ref, b2_ref, w3_ref,
                 o_ref):
    dinv = dv_ref[...]
    zlo = dinv * (slo_ref[...] + ylo_ref[...])
    zhi = dinv * (shi_ref[...] + yhi_ref[...])
    h2 = jnp.maximum(jnp.concatenate([zlo, zhi], axis=1) + b2_ref[0], 0.0)
    o_ref[...] = jnp.dot(h2, w3_ref[...], preferred_element_type=F32) * dinv


def _stage3_body(sa_ref, sb_ref, y3_ref, dv_ref, b3_ref, fcw_ref, z_ref):
    h3 = jnp.maximum(
        dv_ref[...] * (sa_ref[...] + sb_ref[...] + y3_ref[...]) + b3_ref[0],
        0.0)
    z = jnp.dot(h3, fcw_ref[...], preferred_element_type=F32)
    z_ref[...] = jnp.broadcast_to(z, (RB, 32))


def _nspec(w=32):
    return pl.BlockSpec((RB, w), lambda i: (i, 0))


def _fspec(shape):
    return pl.BlockSpec(shape, lambda i: tuple(0 for _ in shape))


_prep = pl.pallas_call(
    _prep_body,
    grid=(NBLK,),
    in_specs=[_nspec(64), _nspec(), _nspec()],
    out_specs=[_nspec(), _nspec(), _nspec()],
    out_shape=[_NSD, _NSD, _NSD],
)

_stage1 = pl.pallas_call(
    _stage1_body,
    grid=(NBLK,),
    in_specs=[_nspec(), _nspec(), _nspec(), _nspec(), _nspec(),
              _fspec((64, 128)), _fspec((1, 128)), _fspec((128, 64))],
    out_specs=[_nspec(), _nspec()],
    out_shape=[_NSD, _NSD],
)

_stage2 = pl.pallas_call(
    _stage2_body,
    grid=(NBLK,),
    in_specs=[_nspec(), _nspec(), _nspec(), _nspec(), _nspec(),
              _fspec((1, 64)), _fspec((64, 32))],
    out_specs=_nspec(),
    out_shape=_NSD,
)

_stage3 = pl.pallas_call(
    _stage3_body,
    grid=(NBLK,),
    in_specs=[_nspec(), _nspec(), _nspec(), _nspec(),
              _fspec((1, 32)), _fspec((32, 1))],
    out_specs=_nspec(),
    out_shape=_NSD,
)


def _pad_nodes(x):
    return jnp.pad(x, ((0, NP - N), (0, 0)))


def _pad_edges(e):
    return jnp.pad(e, (0, EPAD - E), constant_values=NP - 1)


def _encode_graph(x, src, dst, zblk, onesb, W1, b1, W2, b2, W3, b3, fcw):
    da, db = _deg_g(dst, onesb, zblk)
    ylo, yhi, dv = _prep(_pad_nodes(x), da, db)
    s1lo, s1hi = _agg2_g(ylo, yhi, src, dst, zblk)
    y2lo, y2hi = _stage1(s1lo, s1hi, ylo, yhi, dv, W1, b1.reshape(1, 128), W2)
    s2lo, s2hi = _agg2_g(y2lo, y2hi, src, dst, zblk)
    y3 = _stage2(s2lo, s2hi, y2lo, y2hi, dv, b2.reshape(1, 64), W3)
    s3a, s3b = _agg1_g(y3, src, dst, zblk)
    return _stage3(s3a, s3b, y3, dv, b3.reshape(1, 32), fcw)


def kernel(x_q, edge_index_q, batch_q, x_r, edge_index_r, batch_r,
           W1, b1, W2, b2, W3, b3, fcW, fcb):
    ZBLK = jnp.zeros((TILE_N, 32), F32)
    ONESB = jnp.ones((EBLK, 32), F32)
    FCB = jnp.broadcast_to(fcb, (16,))
    BQ = jnp.pad(batch_q, (0, NP - N), constant_values=G)
    BR = jnp.pad(batch_r, (0, NP - N), constant_values=G)
    zbq = _encode_graph(x_q, _pad_edges(edge_index_q[0]),
                        _pad_edges(edge_index_q[1]), ZBLK, ONESB,
                        W1, b1, W2, b2, W3, b3, fcW[:32])
    zbr = _encode_graph(x_r, _pad_edges(edge_index_r[0]),
                        _pad_edges(edge_index_r[1]), ZBLK, ONESB,
                        W1, b1, W2, b2, W3, b3, fcW[32:])
    return _pool_kernel(zbq, zbr, BQ, BR, FCB)
